# Initial kernel scaffold; baseline (speedup 1.0000x reference)
import jax
import jax.numpy as jnp
from jax.experimental import pallas as pl


def kernel(x, shared_w1, shared_w2, shared_w3, routed_w1, routed_w2, routed_w3, router_w, expert_bias):
    raise NotImplementedError("write your pallas kernel here")


# speedup vs baseline: 1.6030x; 1.6030x over previous
import functools
import jax
import jax.numpy as jnp
from jax.experimental import pallas as pl
from jax.experimental.pallas import tpu as pltpu

T = 4096
D = 1024
E = 16
HS = 4096
HR = 1024


def _dotT(a, b):
    return jax.lax.dot_general(a, b, (((1,), (1,)), ((), ())),
                               preferred_element_type=jnp.float32)


def _router_body(x_ref, rw_ref, bias_ref, gates_ref):
    x = x_ref[...]
    rw = rw_ref[...]
    scores = jax.nn.sigmoid(_dotT(x, rw))
    sel = scores + bias_ref[...]
    cols = jax.lax.broadcasted_iota(jnp.int32, (T, E), 1)
    m1 = jnp.max(sel, axis=1, keepdims=True)
    i1 = jnp.min(jnp.where(sel == m1, cols, E), axis=1, keepdims=True)
    mask1 = cols == i1
    sel2 = jnp.where(mask1, -jnp.inf, sel)
    m2 = jnp.max(sel2, axis=1, keepdims=True)
    i2 = jnp.min(jnp.where(sel2 == m2, cols, E), axis=1, keepdims=True)
    mask = mask1 | (cols == i2)
    gates_ref[...] = jnp.where(mask, scores, 0.0)


def _expert_body(gates_ref, x_ref, w1_ref, w2_ref, w3_ref, out_ref):
    e = pl.program_id(1)
    x = x_ref[...]
    h = jax.nn.silu(_dotT(x, w1_ref[0])) * _dotT(x, w2_ref[0])
    res = _dotT(h, w3_ref[0])
    onehot = (jax.lax.broadcasted_iota(jnp.int32, (E, 1), 0) == e).astype(jnp.float32)
    g = jax.lax.dot_general(gates_ref[...], onehot, (((1,), (0,)), ((), ())),
                            preferred_element_type=jnp.float32)
    contrib = res * g

    @pl.when(e == 0)
    def _():
        out_ref[...] = contrib

    @pl.when(e != 0)
    def _():
        out_ref[...] += contrib


def _shared_body(x_ref, w1_ref, w2_ref, w3_ref, routed_ref, out_ref):
    j = pl.program_id(1)
    x = x_ref[...]
    h = jax.nn.silu(_dotT(x, w1_ref[...])) * _dotT(x, w2_ref[...])
    part = _dotT(h, w3_ref[...])

    @pl.when(j == 0)
    def _():
        out_ref[...] = routed_ref[...] + part

    @pl.when(j != 0)
    def _():
        out_ref[...] += part


def kernel(x, shared_w1, shared_w2, shared_w3, routed_w1, routed_w2, routed_w3,
           router_w, expert_bias):
    b, s, d = x.shape
    x2 = x.reshape(T, D)

    gates = pl.pallas_call(
        _router_body,
        out_shape=jax.ShapeDtypeStruct((T, E), jnp.float32),
    )(x2, router_w, expert_bias.reshape(1, E))

    BT = 1024
    routed = pl.pallas_call(
        _expert_body,
        grid=(T // BT, E),
        in_specs=[
            pl.BlockSpec((BT, E), lambda t, e: (t, 0)),
            pl.BlockSpec((BT, D), lambda t, e: (t, 0)),
            pl.BlockSpec((1, HR, D), lambda t, e: (e, 0, 0)),
            pl.BlockSpec((1, HR, D), lambda t, e: (e, 0, 0)),
            pl.BlockSpec((1, D, HR), lambda t, e: (e, 0, 0)),
        ],
        out_specs=pl.BlockSpec((BT, D), lambda t, e: (t, 0)),
        out_shape=jax.ShapeDtypeStruct((T, D), jnp.float32),
    )(gates, x2, routed_w1, routed_w2, routed_w3)

    BS = 512
    HB = 1024
    out = pl.pallas_call(
        _shared_body,
        grid=(T // BS, HS // HB),
        in_specs=[
            pl.BlockSpec((BS, D), lambda t, j: (t, 0)),
            pl.BlockSpec((HB, D), lambda t, j: (j, 0)),
            pl.BlockSpec((HB, D), lambda t, j: (j, 0)),
            pl.BlockSpec((D, HB), lambda t, j: (0, j)),
            pl.BlockSpec((BS, D), lambda t, j: (t, 0)),
        ],
        out_specs=pl.BlockSpec((BS, D), lambda t, j: (t, 0)),
        out_shape=jax.ShapeDtypeStruct((T, D), jnp.float32),
    )(x2, shared_w1, shared_w2, shared_w3, routed)

    return out.reshape(b, s, d)
